# Initial kernel scaffold; baseline (speedup 1.0000x reference)
import jax
import jax.numpy as jnp
from jax.experimental import pallas as pl


def kernel(emb_ind_0, emb_ind_1, adj_0, adj_1, prop_ind_0, prop_ind_1, labels, params):
    raise NotImplementedError("write your pallas kernel here")


# speedup vs baseline: 8.0101x; 8.0101x over previous
import functools

import jax
import jax.numpy as jnp
from jax import lax
from jax.experimental import pallas as pl
from jax.experimental.pallas import tpu as pltpu
from jax.experimental.pallas import tpu_sc as plsc

N = 50000
HID = 64
HALF = 32
EMB_D = 100
EMB_P = 112
B = 1024
E = 800000

CHUNK = 128
IB = 8
NBLK = 49
CH_PER_TILE = NBLK * IB
E_PAD = 16 * CH_PER_TILE * CHUNK
NACC = 50176
ZROWS = 16
OROWS = N // 16
GE_PAD = 51200
GCH = 64

R = 1000
G = N // R

_mesh = plsc.VectorSubcoreMesh(core_axis_name="c", subcore_axis_name="s")
_sc_params = pltpu.CompilerParams(use_tc_tiling_on_sc=False)


@functools.partial(
    pl.kernel,
    out_type=jax.ShapeDtypeStruct((2 * N, HALF), jnp.float32),
    mesh=_mesh,
    compiler_params=_sc_params,
    scratch_types=[
        pltpu.VMEM((IB, CHUNK), jnp.int32),
        pltpu.VMEM((IB, CHUNK), jnp.int32),
        pltpu.VMEM((4, CHUNK, HALF), jnp.float32),
        pltpu.VMEM((ZROWS, HALF), jnp.float32),
        pltpu.VMEM_SHARED((NACC, HALF), jnp.float32),
        [pltpu.SemaphoreType.DMA] * 4,
        [pltpu.SemaphoreType.DMA] * 4,
    ],
)
def _seg_sum(src_hbm, tgt_hbm, hwb_hbm, out_hbm,
             src_blk, tgt_blk, rows_v, zbuf, acc, gsem, ssem):
    c = lax.axis_index("c")
    s = lax.axis_index("s")
    zv = jnp.zeros((16,), jnp.float32)

    @pl.loop(0, ZROWS)
    def _zero_zbuf(i):
        zbuf[i, pl.ds(0, 16)] = zv
        zbuf[i, pl.ds(16, 16)] = zv

    zbase = s * 3136

    @pl.loop(0, 3136 // ZROWS)
    def _zero_acc(k):
        pltpu.sync_copy(zbuf, acc.at[pl.ds(zbase + k * ZROWS, ZROWS)])

    plsc.subcore_barrier()

    blk0 = s * NBLK

    def wait_scatter(b):
        pltpu.make_async_copy(rows_v.at[b], acc.at[tgt_blk.at[0]],
                              ssem[b]).wait()

    @pl.loop(0, NBLK)
    def _blk(bi):
        cb = blk0 + bi
        pltpu.sync_copy(src_hbm.at[c, pl.ds(cb * IB, IB)], src_blk)
        pltpu.sync_copy(tgt_hbm.at[pl.ds(cb * IB, IB)], tgt_blk)

        gd = [None] * IB
        for j in range(3):
            @pl.when(bi > 0)
            def _w(j=j):
                wait_scatter(j)
            gd[j] = pltpu.async_copy(hwb_hbm.at[src_blk.at[j]],
                                     rows_v.at[j], gsem[j])
        for j in range(IB):
            b = j % 4
            if j + 3 < IB:
                nb = (j + 3) % 4
                if j + 3 >= 4:
                    wait_scatter(nb)
                else:
                    @pl.when(bi > 0)
                    def _w3():
                        wait_scatter(3)
                gd[j + 3] = pltpu.async_copy(hwb_hbm.at[src_blk.at[j + 3]],
                                             rows_v.at[nb], gsem[nb])
            gd[j].wait()
            pltpu.async_copy(rows_v.at[b], acc.at[tgt_blk.at[j]],
                             ssem[b], add=True)

    for b in range(4):
        wait_scatter(b)
    plsc.subcore_barrier()

    r0 = s * OROWS
    pltpu.sync_copy(acc.at[pl.ds(r0, OROWS)], out_hbm.at[pl.ds(c * N + r0, OROWS)])


@functools.partial(
    pl.kernel,
    out_type=jax.ShapeDtypeStruct((GE_PAD, EMB_P), jnp.float32),
    mesh=_mesh,
    compiler_params=_sc_params,
    scratch_types=[
        pltpu.VMEM((1, GCH), jnp.int32),
        pltpu.VMEM((GCH, EMB_P), jnp.float32),
        pltpu.SemaphoreType.DMA,
    ],
)
def _emb_gather(tab_hbm, idx_hbm, out_hbm, idx_v, rows_v, sem):
    wid = lax.axis_index("s") * 2 + lax.axis_index("c")

    @pl.loop(0, GE_PAD // (32 * GCH))
    def _(j):
        base = wid * (GE_PAD // 32) + j * GCH
        pltpu.sync_copy(idx_hbm.at[pl.ds(base, GCH)], idx_v.at[0])
        pltpu.async_copy(tab_hbm.at[idx_v.at[0]], rows_v, sem).wait()
        pltpu.sync_copy(rows_v, out_hbm.at[pl.ds(base, GCH)])


@functools.partial(
    pl.kernel,
    out_type=jax.ShapeDtypeStruct((B, HID), jnp.float32),
    mesh=_mesh,
    compiler_params=_sc_params,
    scratch_types=[
        pltpu.VMEM((1, B // 32), jnp.int32),
        pltpu.VMEM((B // 32, HID), jnp.float32),
        pltpu.SemaphoreType.DMA,
    ],
)
def _prop_gather(h_hbm, idx_hbm, out_hbm, idx_v, rows_v, sem):
    wid = lax.axis_index("s") * 2 + lax.axis_index("c")
    per = B // 32
    base = wid * per
    pltpu.sync_copy(idx_hbm.at[pl.ds(base, per)], idx_v.at[0])
    pltpu.async_copy(h_hbm.at[idx_v.at[0]], rows_v, sem).wait()
    pltpu.sync_copy(rows_v, out_hbm.at[pl.ds(base, per)])


def _proj_body(ge_ref, pT_ref, pb_ref, wT_ref, wb_ref, h_ref, hwb_ref):
    h = jnp.dot(ge_ref[...], pT_ref[...],
                preferred_element_type=jnp.float32) + pb_ref[...]
    m = jnp.dot(h, wT_ref[...], preferred_element_type=jnp.float32) + wb_ref[...]
    h_ref[...] = h
    hwb_ref[0] = m[:, :HALF]
    hwb_ref[1] = m[:, HALF:]


_proj_call = pl.pallas_call(
    _proj_body,
    grid=(G,),
    in_specs=[
        pl.BlockSpec((R, EMB_P), lambda i: (i, 0)),
        pl.BlockSpec((EMB_P, HID), lambda i: (0, 0)),
        pl.BlockSpec((1, HID), lambda i: (0, 0)),
        pl.BlockSpec((HID, HID), lambda i: (0, 0)),
        pl.BlockSpec((1, HID), lambda i: (0, 0)),
    ],
    out_specs=[
        pl.BlockSpec((R, HID), lambda i: (i, 0)),
        pl.BlockSpec((2, R, HALF), lambda i: (0, i, 0)),
    ],
    out_shape=[
        jax.ShapeDtypeStruct((N, HID), jnp.float32),
        jax.ShapeDtypeStruct((2, N, HALF), jnp.float32),
    ],
)


def _gru_body(inc_ref, h_ref, wih_ref, whh_ref, bih_ref, bhh_ref,
              wnT_ref, bn_ref, hn_ref, hwb_ref):
    x = jnp.concatenate([inc_ref[0], inc_ref[1]], axis=1)
    h = h_ref[...]

    def mm(a, w_ref, k):
        return jnp.dot(a, w_ref[k], preferred_element_type=jnp.float32)

    i_r = mm(x, wih_ref, 0) + bih_ref[0:1]
    i_z = mm(x, wih_ref, 1) + bih_ref[1:2]
    i_n = mm(x, wih_ref, 2) + bih_ref[2:3]
    h_r = mm(h, whh_ref, 0) + bhh_ref[0:1]
    h_z = mm(h, whh_ref, 1) + bhh_ref[1:2]
    h_n = mm(h, whh_ref, 2) + bhh_ref[2:3]
    r = jax.nn.sigmoid(i_r + h_r)
    z = jax.nn.sigmoid(i_z + h_z)
    n = jnp.tanh(i_n + r * h_n)
    hn = (1.0 - z) * n + z * h
    hn_ref[...] = hn
    m = jnp.dot(hn, wnT_ref[...], preferred_element_type=jnp.float32) + bn_ref[...]
    hwb_ref[0] = m[:, :HALF]
    hwb_ref[1] = m[:, HALF:]


_gru_call = pl.pallas_call(
    _gru_body,
    grid=(G,),
    in_specs=[
        pl.BlockSpec((2, R, HALF), lambda i: (0, i, 0)),
        pl.BlockSpec((R, HID), lambda i: (i, 0)),
        pl.BlockSpec((3, HID, HID), lambda i: (0, 0, 0)),
        pl.BlockSpec((3, HID, HID), lambda i: (0, 0, 0)),
        pl.BlockSpec((3, HID), lambda i: (0, 0)),
        pl.BlockSpec((3, HID), lambda i: (0, 0)),
        pl.BlockSpec((HID, HID), lambda i: (0, 0)),
        pl.BlockSpec((1, HID), lambda i: (0, 0)),
    ],
    out_specs=[
        pl.BlockSpec((R, HID), lambda i: (i, 0)),
        pl.BlockSpec((2, R, HALF), lambda i: (0, i, 0)),
    ],
    out_shape=[
        jax.ShapeDtypeStruct((N, HID), jnp.float32),
        jax.ShapeDtypeStruct((2, N, HALF), jnp.float32),
    ],
)


def _cla_body(g0_ref, g1_ref, w1a_ref, w1b_ref, b1_ref, w2_ref, b2_ref,
              y_ref, logit_ref, loss_ref):
    hc = jax.nn.relu(
        jnp.dot(g0_ref[...], w1a_ref[...], preferred_element_type=jnp.float32)
        + jnp.dot(g1_ref[...], w1b_ref[...], preferred_element_type=jnp.float32)
        + b1_ref[...])
    ll = jnp.sum(hc * w2_ref[...], axis=1, keepdims=True) + b2_ref[...]
    lg = jax.nn.sigmoid(ll)
    p = jnp.clip(lg, 1e-7, 1.0 - 1e-7)
    y = y_ref[...]
    lv = y * jnp.log(p) + (1.0 - y) * jnp.log(1.0 - p)
    logit_ref[...] = lg
    loss_ref[...] = jnp.reshape(-jnp.sum(lv) / B, (1, 1))


_cla_call = pl.pallas_call(
    _cla_body,
    grid=(1,),
    in_specs=[
        pl.BlockSpec((B, HID), lambda i: (0, 0)),
        pl.BlockSpec((B, HID), lambda i: (0, 0)),
        pl.BlockSpec((HID, HID), lambda i: (0, 0)),
        pl.BlockSpec((HID, HID), lambda i: (0, 0)),
        pl.BlockSpec((1, HID), lambda i: (0, 0)),
        pl.BlockSpec((1, HID), lambda i: (0, 0)),
        pl.BlockSpec((1, 1), lambda i: (0, 0)),
        pl.BlockSpec((B, 1), lambda i: (0, 0)),
    ],
    out_specs=[
        pl.BlockSpec((B, 1), lambda i: (0, 0)),
        pl.BlockSpec((1, 1), lambda i: (0, 0)),
    ],
    out_shape=[
        jax.ShapeDtypeStruct((B, 1), jnp.float32),
        jax.ShapeDtypeStruct((1, 1), jnp.float32),
    ],
)


def kernel(emb_ind_0, emb_ind_1, adj_0, adj_1, prop_ind_0, prop_ind_1,
           labels, params):
    p = params
    tab = jnp.pad(p["emb"], ((0, 0), (0, EMB_P - EMB_D)))
    projT = jnp.pad(p["proj_W"].T, ((0, EMB_P - EMB_D), (0, 0)))
    pb = p["proj_b"].reshape(1, HID)
    msgT = [p["msg_W"][l].T for l in range(2)]
    msgb = [p["msg_b"][l].reshape(1, HID) for l in range(2)]
    wih3 = [p["gru_Wih"][l].reshape(3, HID, HID).transpose(0, 2, 1)
            for l in range(2)]
    whh3 = [p["gru_Whh"][l].reshape(3, HID, HID).transpose(0, 2, 1)
            for l in range(2)]
    bih3 = [p["gru_bih"][l].reshape(3, HID) for l in range(2)]
    bhh3 = [p["gru_bhh"][l].reshape(3, HID) for l in range(2)]

    layer_seq = (0, 0, 0, 1, 1, 1)
    ge_out = []
    for emb_ind, adj, prop_ind in ((emb_ind_0, adj_0, prop_ind_0),
                                   (emb_ind_1, adj_1, prop_ind_1)):
        idx_pad = jnp.concatenate(
            [emb_ind.astype(jnp.int32), jnp.zeros((GE_PAD - N,), jnp.int32)])
        ge_raw = _emb_gather(tab, idx_pad)
        h, hwb = _proj_call(ge_raw, projT, pb, msgT[0], msgb[0])

        src = adj[:, 0].astype(jnp.int32)
        tgt = adj[:, 1].astype(jnp.int32)
        srcp = jnp.concatenate([src, jnp.zeros((E_PAD - E,), jnp.int32)])
        src2 = jnp.stack([srcp, srcp + N]).reshape(2, E_PAD // CHUNK, CHUNK)
        tgtp = jnp.concatenate(
            [tgt, jnp.full((E_PAD - E,), N, jnp.int32)]
        ).reshape(E_PAD // CHUNK, CHUNK)

        for t in range(6):
            l = layer_seq[t]
            ln = layer_seq[t + 1] if t < 5 else 1
            inc = _seg_sum(src2, tgtp, hwb.reshape(2 * N, HALF))
            h, hwb = _gru_call(inc.reshape(2, N, HALF), h,
                               wih3[l], whh3[l], bih3[l], bhh3[l],
                               msgT[ln], msgb[ln])
        ge_out.append(_prop_gather(h, prop_ind.astype(jnp.int32)))

    w1aT = p["cla_W1"][:, :HID].T
    w1bT = p["cla_W1"][:, HID:].T
    b1 = p["cla_b1"].reshape(1, HID)
    w2 = p["cla_W2"].reshape(1, HID)
    b2 = p["cla_b2"].reshape(1, 1)
    y = labels.astype(jnp.float32).reshape(B, 1)
    logits, loss = _cla_call(ge_out[0], ge_out[1], w1aT, w1bT, b1, w2, b2, y)
    return logits, loss[0, 0]


# speedup vs baseline: 8.0786x; 1.0086x over previous
import functools

import jax
import jax.numpy as jnp
from jax import lax
from jax.experimental import pallas as pl
from jax.experimental.pallas import tpu as pltpu
from jax.experimental.pallas import tpu_sc as plsc

N = 50000
HID = 64
HALF = 32
EMB_D = 100
EMB_P = 112
B = 1024
E = 800000

CHUNK = 128
IB = 8
NBLK = 49
CH_PER_TILE = NBLK * IB
E_PAD = 16 * CH_PER_TILE * CHUNK
NACC = 50176
ZROWS = 16
OROWS = N // 16
GE_PAD = 51200
GCH = 64

R = 1000
G = N // R

_mesh = plsc.VectorSubcoreMesh(core_axis_name="c", subcore_axis_name="s")
_sc_params = pltpu.CompilerParams(use_tc_tiling_on_sc=False)


@functools.partial(
    pl.kernel,
    out_type=jax.ShapeDtypeStruct((2 * N, HALF), jnp.float32),
    mesh=_mesh,
    compiler_params=_sc_params,
    scratch_types=[
        pltpu.VMEM((IB, CHUNK), jnp.int32),
        pltpu.VMEM((IB, CHUNK), jnp.int32),
        pltpu.VMEM((5, CHUNK, HALF), jnp.float32),
        pltpu.VMEM((ZROWS, HALF), jnp.float32),
        pltpu.VMEM_SHARED((NACC, HALF), jnp.float32),
        [pltpu.SemaphoreType.DMA] * 5,
        [pltpu.SemaphoreType.DMA] * 5,
    ],
)
def _seg_sum(src_hbm, tgt_hbm, hwb_hbm, out_hbm,
             src_blk, tgt_blk, rows_v, zbuf, acc, gsem, ssem):
    c = lax.axis_index("c")
    s = lax.axis_index("s")
    zv = jnp.zeros((16,), jnp.float32)

    @pl.loop(0, ZROWS)
    def _zero_zbuf(i):
        zbuf[i, pl.ds(0, 16)] = zv
        zbuf[i, pl.ds(16, 16)] = zv

    zbase = s * 3136

    @pl.loop(0, 3136 // ZROWS)
    def _zero_acc(k):
        pltpu.sync_copy(zbuf, acc.at[pl.ds(zbase + k * ZROWS, ZROWS)])

    plsc.subcore_barrier()

    blk0 = s * NBLK

    def wait_scatter(b):
        pltpu.make_async_copy(rows_v.at[b], acc.at[tgt_blk.at[0]],
                              ssem[b]).wait()

    @pl.loop(0, NBLK)
    def _blk(bi):
        cb = blk0 + bi
        pltpu.sync_copy(src_hbm.at[c, pl.ds(cb * IB, IB)], src_blk)
        pltpu.sync_copy(tgt_hbm.at[pl.ds(cb * IB, IB)], tgt_blk)

        gd = [None] * IB
        for j in range(4):
            @pl.when(bi > 0)
            def _w(j=j):
                wait_scatter(j)
            gd[j] = pltpu.async_copy(hwb_hbm.at[src_blk.at[j]],
                                     rows_v.at[j], gsem[j])
        for j in range(IB):
            b = j % 5
            if j + 4 < IB:
                nb = (j + 4) % 5
                if j + 4 >= 5:
                    wait_scatter(nb)
                else:
                    @pl.when(bi > 0)
                    def _w4():
                        wait_scatter(4)
                gd[j + 4] = pltpu.async_copy(hwb_hbm.at[src_blk.at[j + 4]],
                                             rows_v.at[nb], gsem[nb])
            gd[j].wait()
            pltpu.async_copy(rows_v.at[b], acc.at[tgt_blk.at[j]],
                             ssem[b], add=True)

    for b in range(5):
        wait_scatter(b)
    plsc.subcore_barrier()

    r0 = s * OROWS
    pltpu.sync_copy(acc.at[pl.ds(r0, OROWS)], out_hbm.at[pl.ds(c * N + r0, OROWS)])


@functools.partial(
    pl.kernel,
    out_type=jax.ShapeDtypeStruct((GE_PAD, EMB_P), jnp.float32),
    mesh=_mesh,
    compiler_params=_sc_params,
    scratch_types=[
        pltpu.VMEM((1, GCH), jnp.int32),
        pltpu.VMEM((GCH, EMB_P), jnp.float32),
        pltpu.SemaphoreType.DMA,
    ],
)
def _emb_gather(tab_hbm, idx_hbm, out_hbm, idx_v, rows_v, sem):
    wid = lax.axis_index("s") * 2 + lax.axis_index("c")

    @pl.loop(0, GE_PAD // (32 * GCH))
    def _(j):
        base = wid * (GE_PAD // 32) + j * GCH
        pltpu.sync_copy(idx_hbm.at[pl.ds(base, GCH)], idx_v.at[0])
        pltpu.async_copy(tab_hbm.at[idx_v.at[0]], rows_v, sem).wait()
        pltpu.sync_copy(rows_v, out_hbm.at[pl.ds(base, GCH)])


@functools.partial(
    pl.kernel,
    out_type=jax.ShapeDtypeStruct((B, HID), jnp.float32),
    mesh=_mesh,
    compiler_params=_sc_params,
    scratch_types=[
        pltpu.VMEM((1, B // 32), jnp.int32),
        pltpu.VMEM((B // 32, HID), jnp.float32),
        pltpu.SemaphoreType.DMA,
    ],
)
def _prop_gather(h_hbm, idx_hbm, out_hbm, idx_v, rows_v, sem):
    wid = lax.axis_index("s") * 2 + lax.axis_index("c")
    per = B // 32
    base = wid * per
    pltpu.sync_copy(idx_hbm.at[pl.ds(base, per)], idx_v.at[0])
    pltpu.async_copy(h_hbm.at[idx_v.at[0]], rows_v, sem).wait()
    pltpu.sync_copy(rows_v, out_hbm.at[pl.ds(base, per)])


def _proj_body(ge_ref, pT_ref, pb_ref, wT_ref, wb_ref, h_ref, hwb_ref):
    h = jnp.dot(ge_ref[...], pT_ref[...],
                preferred_element_type=jnp.float32) + pb_ref[...]
    m = jnp.dot(h, wT_ref[...], preferred_element_type=jnp.float32) + wb_ref[...]
    h_ref[...] = h
    hwb_ref[0] = m[:, :HALF]
    hwb_ref[1] = m[:, HALF:]


_proj_call = pl.pallas_call(
    _proj_body,
    grid=(G,),
    in_specs=[
        pl.BlockSpec((R, EMB_P), lambda i: (i, 0)),
        pl.BlockSpec((EMB_P, HID), lambda i: (0, 0)),
        pl.BlockSpec((1, HID), lambda i: (0, 0)),
        pl.BlockSpec((HID, HID), lambda i: (0, 0)),
        pl.BlockSpec((1, HID), lambda i: (0, 0)),
    ],
    out_specs=[
        pl.BlockSpec((R, HID), lambda i: (i, 0)),
        pl.BlockSpec((2, R, HALF), lambda i: (0, i, 0)),
    ],
    out_shape=[
        jax.ShapeDtypeStruct((N, HID), jnp.float32),
        jax.ShapeDtypeStruct((2, N, HALF), jnp.float32),
    ],
)


def _gru_body(inc_ref, h_ref, wih_ref, whh_ref, bih_ref, bhh_ref,
              wnT_ref, bn_ref, hn_ref, hwb_ref):
    x = jnp.concatenate([inc_ref[0], inc_ref[1]], axis=1)
    h = h_ref[...]

    def mm(a, w_ref, k):
        return jnp.dot(a, w_ref[k], preferred_element_type=jnp.float32)

    i_r = mm(x, wih_ref, 0) + bih_ref[0:1]
    i_z = mm(x, wih_ref, 1) + bih_ref[1:2]
    i_n = mm(x, wih_ref, 2) + bih_ref[2:3]
    h_r = mm(h, whh_ref, 0) + bhh_ref[0:1]
    h_z = mm(h, whh_ref, 1) + bhh_ref[1:2]
    h_n = mm(h, whh_ref, 2) + bhh_ref[2:3]
    r = jax.nn.sigmoid(i_r + h_r)
    z = jax.nn.sigmoid(i_z + h_z)
    n = jnp.tanh(i_n + r * h_n)
    hn = (1.0 - z) * n + z * h
    hn_ref[...] = hn
    m = jnp.dot(hn, wnT_ref[...], preferred_element_type=jnp.float32) + bn_ref[...]
    hwb_ref[0] = m[:, :HALF]
    hwb_ref[1] = m[:, HALF:]


_gru_call = pl.pallas_call(
    _gru_body,
    grid=(G,),
    in_specs=[
        pl.BlockSpec((2, R, HALF), lambda i: (0, i, 0)),
        pl.BlockSpec((R, HID), lambda i: (i, 0)),
        pl.BlockSpec((3, HID, HID), lambda i: (0, 0, 0)),
        pl.BlockSpec((3, HID, HID), lambda i: (0, 0, 0)),
        pl.BlockSpec((3, HID), lambda i: (0, 0)),
        pl.BlockSpec((3, HID), lambda i: (0, 0)),
        pl.BlockSpec((HID, HID), lambda i: (0, 0)),
        pl.BlockSpec((1, HID), lambda i: (0, 0)),
    ],
    out_specs=[
        pl.BlockSpec((R, HID), lambda i: (i, 0)),
        pl.BlockSpec((2, R, HALF), lambda i: (0, i, 0)),
    ],
    out_shape=[
        jax.ShapeDtypeStruct((N, HID), jnp.float32),
        jax.ShapeDtypeStruct((2, N, HALF), jnp.float32),
    ],
)


def _cla_body(g0_ref, g1_ref, w1a_ref, w1b_ref, b1_ref, w2_ref, b2_ref,
              y_ref, logit_ref, loss_ref):
    hc = jax.nn.relu(
        jnp.dot(g0_ref[...], w1a_ref[...], preferred_element_type=jnp.float32)
        + jnp.dot(g1_ref[...], w1b_ref[...], preferred_element_type=jnp.float32)
        + b1_ref[...])
    ll = jnp.sum(hc * w2_ref[...], axis=1, keepdims=True) + b2_ref[...]
    lg = jax.nn.sigmoid(ll)
    p = jnp.clip(lg, 1e-7, 1.0 - 1e-7)
    y = y_ref[...]
    lv = y * jnp.log(p) + (1.0 - y) * jnp.log(1.0 - p)
    logit_ref[...] = lg
    loss_ref[...] = jnp.reshape(-jnp.sum(lv) / B, (1, 1))


_cla_call = pl.pallas_call(
    _cla_body,
    grid=(1,),
    in_specs=[
        pl.BlockSpec((B, HID), lambda i: (0, 0)),
        pl.BlockSpec((B, HID), lambda i: (0, 0)),
        pl.BlockSpec((HID, HID), lambda i: (0, 0)),
        pl.BlockSpec((HID, HID), lambda i: (0, 0)),
        pl.BlockSpec((1, HID), lambda i: (0, 0)),
        pl.BlockSpec((1, HID), lambda i: (0, 0)),
        pl.BlockSpec((1, 1), lambda i: (0, 0)),
        pl.BlockSpec((B, 1), lambda i: (0, 0)),
    ],
    out_specs=[
        pl.BlockSpec((B, 1), lambda i: (0, 0)),
        pl.BlockSpec((1, 1), lambda i: (0, 0)),
    ],
    out_shape=[
        jax.ShapeDtypeStruct((B, 1), jnp.float32),
        jax.ShapeDtypeStruct((1, 1), jnp.float32),
    ],
)


def kernel(emb_ind_0, emb_ind_1, adj_0, adj_1, prop_ind_0, prop_ind_1,
           labels, params):
    p = params
    tab = jnp.pad(p["emb"], ((0, 0), (0, EMB_P - EMB_D)))
    projT = jnp.pad(p["proj_W"].T, ((0, EMB_P - EMB_D), (0, 0)))
    pb = p["proj_b"].reshape(1, HID)
    msgT = [p["msg_W"][l].T for l in range(2)]
    msgb = [p["msg_b"][l].reshape(1, HID) for l in range(2)]
    wih3 = [p["gru_Wih"][l].reshape(3, HID, HID).transpose(0, 2, 1)
            for l in range(2)]
    whh3 = [p["gru_Whh"][l].reshape(3, HID, HID).transpose(0, 2, 1)
            for l in range(2)]
    bih3 = [p["gru_bih"][l].reshape(3, HID) for l in range(2)]
    bhh3 = [p["gru_bhh"][l].reshape(3, HID) for l in range(2)]

    layer_seq = (0, 0, 0, 1, 1, 1)
    ge_out = []
    for emb_ind, adj, prop_ind in ((emb_ind_0, adj_0, prop_ind_0),
                                   (emb_ind_1, adj_1, prop_ind_1)):
        idx_pad = jnp.concatenate(
            [emb_ind.astype(jnp.int32), jnp.zeros((GE_PAD - N,), jnp.int32)])
        ge_raw = _emb_gather(tab, idx_pad)
        h, hwb = _proj_call(ge_raw, projT, pb, msgT[0], msgb[0])

        src = adj[:, 0].astype(jnp.int32)
        tgt = adj[:, 1].astype(jnp.int32)
        srcp = jnp.concatenate([src, jnp.zeros((E_PAD - E,), jnp.int32)])
        src2 = jnp.stack([srcp, srcp + N]).reshape(2, E_PAD // CHUNK, CHUNK)
        tgtp = jnp.concatenate(
            [tgt, jnp.full((E_PAD - E,), N, jnp.int32)]
        ).reshape(E_PAD // CHUNK, CHUNK)

        for t in range(6):
            l = layer_seq[t]
            ln = layer_seq[t + 1] if t < 5 else 1
            inc = _seg_sum(src2, tgtp, hwb.reshape(2 * N, HALF))
            h, hwb = _gru_call(inc.reshape(2, N, HALF), h,
                               wih3[l], whh3[l], bih3[l], bhh3[l],
                               msgT[ln], msgb[ln])
        ge_out.append(_prop_gather(h, prop_ind.astype(jnp.int32)))

    w1aT = p["cla_W1"][:, :HID].T
    w1bT = p["cla_W1"][:, HID:].T
    b1 = p["cla_b1"].reshape(1, HID)
    w2 = p["cla_W2"].reshape(1, HID)
    b2 = p["cla_b2"].reshape(1, 1)
    y = labels.astype(jnp.float32).reshape(B, 1)
    logits, loss = _cla_call(ge_out[0], ge_out[1], w1aT, w1bT, b1, w2, b2, y)
    return logits, loss[0, 0]
